# Initial kernel scaffold; baseline (speedup 1.0000x reference)
import jax
import jax.numpy as jnp
from jax.experimental import pallas as pl


def kernel(x, adj, weight, bias):
    raise NotImplementedError("write your pallas kernel here")


# speedup vs baseline: 6.4341x; 6.4341x over previous
import functools

import jax
import jax.numpy as jnp
from jax import lax
from jax.experimental import pallas as pl
from jax.experimental.pallas import tpu as pltpu
from jax.experimental.pallas import tpu_sc as plsc

MIN_NORM = 1e-15
EPS = 4e-3
N, E, D = 10000, 320000, 128
C = 1.0

NC, NS = 2, 16
NW = NC * NS
EPW = E // NW
K = 80
NCHUNK = EPW // K
ROWS_PER_TILE = 632
NPAD = NS * ROWS_PER_TILE


def _artanh(x):
    x = jnp.clip(x, -1.0 + 1e-7, 1.0 - 1e-7)
    return 0.5 * (jnp.log1p(x) - jnp.log1p(-x))


def _norm(x):
    return jnp.clip(
        jnp.sqrt(jnp.sum(x * x, axis=-1, keepdims=True)), MIN_NORM, None)


def _proj(x):
    norm = _norm(x)
    maxnorm = (1.0 - EPS)
    return jnp.where(norm > maxnorm, x / norm * maxnorm, x)


def _expmap0(u):
    u_norm = _norm(u)
    return jnp.tanh(u_norm) * u / u_norm


def _logmap0(p):
    p_norm = _norm(p)
    return _artanh(p_norm) * p / p_norm


def _mobius_add(x, y):
    x2 = jnp.sum(x * x, axis=-1, keepdims=True)
    y2 = jnp.sum(y * y, axis=-1, keepdims=True)
    xy = jnp.sum(x * y, axis=-1, keepdims=True)
    num = (1.0 + 2.0 * xy + y2) * x + (1.0 - x2) * y
    denom = 1.0 + 2.0 * xy + x2 * y2
    return num / jnp.clip(denom, MIN_NORM, None)


def _tc_pre_body(x_ref, w_ref, b_ref, o_ref):
    x = x_ref[...]
    w = w_ref[...]
    mx = lax.dot_general(
        x, w, (((1,), (1,)), ((), ())),
        preferred_element_type=jnp.float32,
        precision=lax.Precision.HIGHEST)
    x_norm = _norm(x)
    mx_norm = _norm(mx)
    res_c = jnp.tanh(mx_norm / x_norm * _artanh(x_norm)) * mx / mx_norm
    cond = jnp.all(mx == 0.0, axis=-1, keepdims=True)
    res = _proj(jnp.where(cond, 0.0, res_c))
    hyp_bias = _proj(_expmap0(b_ref[...]))
    res = _proj(_mobius_add(res, hyp_bias))
    o_ref[...] = _logmap0(res)


def _tc_post_body(p_ref, o_ref):
    agg = p_ref[0] + p_ref[1]
    h = _proj(_expmap0(agg))
    xt2 = jax.nn.relu(_logmap0(h))
    o_ref[...] = _proj(_expmap0(xt2))


def _sc_agg_body(xt_hbm, s_hbm, r_hbm, z_hbm, out_hbm,
                 sidx_v, ridx_v, rows_v, acc_sh):
    cid = lax.axis_index("c")
    sid = lax.axis_index("s")
    wid = sid * NC + cid

    row0 = sid * ROWS_PER_TILE
    pltpu.sync_copy(z_hbm.at[pl.ds(0, ROWS_PER_TILE)],
                    acc_sh.at[pl.ds(row0, ROWS_PER_TILE)])

    pltpu.sync_copy(s_hbm.at[wid], sidx_v)
    pltpu.sync_copy(r_hbm.at[wid], ridx_v)

    plsc.subcore_barrier()

    def chunk(i, _):
        pltpu.sync_copy(xt_hbm.at[sidx_v.at[i]], rows_v)
        pltpu.sync_copy(rows_v, acc_sh.at[ridx_v.at[i]], add=True)
        return _

    lax.fori_loop(0, NCHUNK, chunk, None)

    plsc.subcore_barrier()

    pltpu.sync_copy(acc_sh.at[pl.ds(row0, ROWS_PER_TILE)],
                    out_hbm.at[cid].at[pl.ds(row0, ROWS_PER_TILE)])


@functools.cache
def _sc_agg():
    return pl.kernel(
        _sc_agg_body,
        out_type=jax.ShapeDtypeStruct((NC, NPAD, D), jnp.float32),
        mesh=plsc.VectorSubcoreMesh(core_axis_name="c", subcore_axis_name="s",
                                    num_cores=NC, num_subcores=NS),
        scratch_types=[
            pltpu.VMEM((NCHUNK, K), jnp.int32),
            pltpu.VMEM((NCHUNK, K), jnp.int32),
            pltpu.VMEM((K, D), jnp.float32),
            pltpu.VMEM_SHARED((NPAD, D), jnp.float32),
        ],
    )


@jax.jit
def kernel(x, adj, weight, bias):
    blk = 1000
    grid = N // blk

    xt = pl.pallas_call(
        _tc_pre_body,
        grid=(grid,),
        in_specs=[
            pl.BlockSpec((blk, D), lambda i: (i, 0)),
            pl.BlockSpec((D, D), lambda i: (0, 0)),
            pl.BlockSpec((1, D), lambda i: (0, 0)),
        ],
        out_specs=pl.BlockSpec((blk, D), lambda i: (i, 0)),
        out_shape=jax.ShapeDtypeStruct((N, D), jnp.float32),
    )(x, weight, bias.reshape(1, D))

    s3 = adj[0].astype(jnp.int32).reshape(NW, NCHUNK, K)
    r3 = adj[1].astype(jnp.int32).reshape(NW, NCHUNK, K)
    zeros = jnp.zeros((ROWS_PER_TILE, D), jnp.float32)

    partials = _sc_agg()(xt, s3, r3, zeros)

    out = pl.pallas_call(
        _tc_post_body,
        grid=(grid,),
        in_specs=[pl.BlockSpec((NC, blk, D), lambda i: (0, i, 0))],
        out_specs=pl.BlockSpec((blk, D), lambda i: (i, 0)),
        out_shape=jax.ShapeDtypeStruct((N, D), jnp.float32),
    )(partials)

    return (out, adj)


# speedup vs baseline: 7.7882x; 1.2105x over previous
import functools

import jax
import jax.numpy as jnp
from jax import lax
from jax.experimental import pallas as pl
from jax.experimental.pallas import tpu as pltpu
from jax.experimental.pallas import tpu_sc as plsc

MIN_NORM = 1e-15
EPS = 4e-3
N, E, D = 10000, 320000, 128
C = 1.0

NC, NS = 2, 16
NW = NC * NS
K = 80
NCHUNK = 125
EPW = NCHUNK * K
EPAD = NW * EPW
ROWS_PER_TILE = 632
NPAD = NS * ROWS_PER_TILE


def _artanh(x):
    x = jnp.clip(x, -1.0 + 1e-7, 1.0 - 1e-7)
    return 0.5 * (jnp.log1p(x) - jnp.log1p(-x))


def _norm(x):
    return jnp.clip(
        jnp.sqrt(jnp.sum(x * x, axis=-1, keepdims=True)), MIN_NORM, None)


def _proj(x):
    norm = _norm(x)
    maxnorm = (1.0 - EPS)
    return jnp.where(norm > maxnorm, x / norm * maxnorm, x)


def _expmap0(u):
    u_norm = _norm(u)
    return jnp.tanh(u_norm) * u / u_norm


def _logmap0(p):
    p_norm = _norm(p)
    return _artanh(p_norm) * p / p_norm


def _mobius_add(x, y):
    x2 = jnp.sum(x * x, axis=-1, keepdims=True)
    y2 = jnp.sum(y * y, axis=-1, keepdims=True)
    xy = jnp.sum(x * y, axis=-1, keepdims=True)
    num = (1.0 + 2.0 * xy + y2) * x + (1.0 - x2) * y
    denom = 1.0 + 2.0 * xy + x2 * y2
    return num / jnp.clip(denom, MIN_NORM, None)


def _tc_pre_body(x_ref, w_ref, b_ref, o_ref):
    x = x_ref[...]
    w = w_ref[...]
    mx = lax.dot_general(
        x, w, (((1,), (1,)), ((), ())),
        preferred_element_type=jnp.float32,
        precision=lax.Precision.HIGHEST)
    x_norm = _norm(x)
    mx_norm = _norm(mx)
    res_c = jnp.tanh(mx_norm / x_norm * _artanh(x_norm)) * mx / mx_norm
    cond = jnp.all(mx == 0.0, axis=-1, keepdims=True)
    res = _proj(jnp.where(cond, 0.0, res_c))
    hyp_bias = _proj(_expmap0(b_ref[...]))
    res = _proj(_mobius_add(res, hyp_bias))
    o_ref[...] = _logmap0(res)


def _tc_post_body(p_ref, o_ref):
    agg = p_ref[0] + p_ref[1]
    h = _proj(_expmap0(agg))
    xt2 = jax.nn.relu(_logmap0(h))
    o_ref[...] = _proj(_expmap0(xt2))


def _sc_agg_body(xt_hbm, s_hbm, r_hbm, z_hbm, out_hbm,
                 sidx_v, ridx_v, rows_v, acc_sh, gsem, ssem):
    cid = lax.axis_index("c")
    sid = lax.axis_index("s")
    wid = sid * NC + cid

    row0 = sid * ROWS_PER_TILE
    pltpu.sync_copy(z_hbm.at[pl.ds(0, ROWS_PER_TILE)],
                    acc_sh.at[pl.ds(row0, ROWS_PER_TILE)])

    pltpu.sync_copy(s_hbm.at[pl.ds(wid * EPW, EPW)], sidx_v)
    pltpu.sync_copy(r_hbm.at[wid], ridx_v)

    plsc.subcore_barrier()

    def slot(i):
        return rows_v.at[pl.ds((i % 2) * K, K)]

    def drain(sem):
        pltpu.make_async_copy(xt_hbm.at[pl.ds(0, K)],
                              rows_v.at[pl.ds(0, K)], sem).wait()

    def step(i, _):
        @pl.when(i >= 2)
        def _():
            drain(ssem)

        @pl.when(i >= 1)
        def _():
            drain(gsem)

        @pl.when(i < NCHUNK)
        def _():
            pltpu.make_async_copy(
                xt_hbm.at[sidx_v.at[pl.ds(i * K, K)]], slot(i), gsem).start()

        @pl.when(i >= 1)
        def _():
            pltpu.make_async_copy(
                slot(i - 1), acc_sh.at[ridx_v.at[i - 1]], ssem).start(add=True)

        return _

    lax.fori_loop(0, NCHUNK + 1, step, None)
    drain(ssem)

    plsc.subcore_barrier()

    pltpu.sync_copy(acc_sh.at[pl.ds(row0, ROWS_PER_TILE)],
                    out_hbm.at[cid].at[pl.ds(row0, ROWS_PER_TILE)])


@functools.cache
def _sc_agg():
    return pl.kernel(
        _sc_agg_body,
        out_type=jax.ShapeDtypeStruct((NC, NPAD, D), jnp.float32),
        mesh=plsc.VectorSubcoreMesh(core_axis_name="c", subcore_axis_name="s",
                                    num_cores=NC, num_subcores=NS),
        scratch_types=[
            pltpu.VMEM((EPW,), jnp.int32),
            pltpu.VMEM((NCHUNK, K), jnp.int32),
            pltpu.VMEM((2 * K, D), jnp.float32),
            pltpu.VMEM_SHARED((NPAD, D), jnp.float32),
            pltpu.SemaphoreType.DMA,
            pltpu.SemaphoreType.DMA,
        ],
    )


@jax.jit
def kernel(x, adj, weight, bias):
    blk = 1000
    grid = N // blk

    xt = pl.pallas_call(
        _tc_pre_body,
        grid=(grid,),
        in_specs=[
            pl.BlockSpec((blk, D), lambda i: (i, 0)),
            pl.BlockSpec((D, D), lambda i: (0, 0)),
            pl.BlockSpec((1, D), lambda i: (0, 0)),
        ],
        out_specs=pl.BlockSpec((blk, D), lambda i: (i, 0)),
        out_shape=jax.ShapeDtypeStruct((N, D), jnp.float32),
    )(x, weight, bias.reshape(1, D))

    s1 = adj[0].astype(jnp.int32)
    r3 = adj[1].astype(jnp.int32).reshape(NW, NCHUNK, K)
    zeros = jnp.zeros((ROWS_PER_TILE, D), jnp.float32)

    partials = _sc_agg()(xt, s1, r3, zeros)

    out = pl.pallas_call(
        _tc_post_body,
        grid=(grid,),
        in_specs=[pl.BlockSpec((NC, blk, D), lambda i: (0, i, 0))],
        out_specs=pl.BlockSpec((blk, D), lambda i: (i, 0)),
        out_shape=jax.ShapeDtypeStruct((N, D), jnp.float32),
    )(partials)

    return (out, adj)


# speedup vs baseline: 9.5238x; 1.2228x over previous
import functools

import jax
import jax.numpy as jnp
from jax import lax
from jax.experimental import pallas as pl
from jax.experimental.pallas import tpu as pltpu
from jax.experimental.pallas import tpu_sc as plsc

MIN_NORM = 1e-15
EPS = 4e-3
N, E, D = 10000, 320000, 128
C = 1.0

NC, NS = 2, 16
NW = NC * NS
K = 80
NCHUNK = 125
EPW = NCHUNK * K
EPAD = NW * EPW
ROWS_PER_TILE = 632
NPAD = NS * ROWS_PER_TILE


def _artanh(x):
    x = jnp.clip(x, -1.0 + 1e-7, 1.0 - 1e-7)
    return 0.5 * (jnp.log1p(x) - jnp.log1p(-x))


def _norm(x):
    return jnp.clip(
        jnp.sqrt(jnp.sum(x * x, axis=-1, keepdims=True)), MIN_NORM, None)


def _proj(x):
    norm = _norm(x)
    maxnorm = (1.0 - EPS)
    return jnp.where(norm > maxnorm, x / norm * maxnorm, x)


def _expmap0(u):
    u_norm = _norm(u)
    return jnp.tanh(u_norm) * u / u_norm


def _logmap0(p):
    p_norm = _norm(p)
    return _artanh(p_norm) * p / p_norm


def _mobius_add(x, y):
    x2 = jnp.sum(x * x, axis=-1, keepdims=True)
    y2 = jnp.sum(y * y, axis=-1, keepdims=True)
    xy = jnp.sum(x * y, axis=-1, keepdims=True)
    num = (1.0 + 2.0 * xy + y2) * x + (1.0 - x2) * y
    denom = 1.0 + 2.0 * xy + x2 * y2
    return num / jnp.clip(denom, MIN_NORM, None)


def _tc_pre_body(x_ref, w_ref, b_ref, o_ref):
    x = x_ref[...]
    w = w_ref[...]
    mx = lax.dot_general(
        x, w, (((1,), (1,)), ((), ())),
        preferred_element_type=jnp.float32,
        precision=lax.Precision.HIGHEST)
    x_norm = _norm(x)
    mx_norm = _norm(mx)
    res_c = jnp.tanh(mx_norm / x_norm * _artanh(x_norm)) * mx / mx_norm
    cond = jnp.all(mx == 0.0, axis=-1, keepdims=True)
    res = _proj(jnp.where(cond, 0.0, res_c))
    hyp_bias = _proj(_expmap0(b_ref[...]))
    res = _proj(_mobius_add(res, hyp_bias))
    o_ref[...] = _logmap0(res)


def _tc_post_body(p_ref, o_ref):
    agg = p_ref[0] + p_ref[1]
    h = _proj(_expmap0(agg))
    xt2 = jax.nn.relu(_logmap0(h))
    o_ref[...] = _proj(_expmap0(xt2))


def _sc_agg_body(xt_hbm, i_hbm, z_hbm, out_hbm,
                 idx_v, rows_v, acc_sh, gsem, ssem, isem):
    cid = lax.axis_index("c")
    sid = lax.axis_index("s")
    wid = sid * NC + cid

    row0 = sid * ROWS_PER_TILE
    pltpu.sync_copy(z_hbm.at[pl.ds(0, ROWS_PER_TILE)],
                    acc_sh.at[pl.ds(row0, ROWS_PER_TILE)])

    plsc.subcore_barrier()

    def slot(i):
        return rows_v.at[pl.ds((i % 4) * K, K)]

    def drain_rows(sem):
        pltpu.make_async_copy(xt_hbm.at[pl.ds(0, K)],
                              rows_v.at[pl.ds(0, K)], sem).wait()

    def drain_idx(sem):
        pltpu.make_async_copy(i_hbm.at[wid, 0], idx_v.at[0], sem).wait()

    def step(i, _):
        p = i % 2

        @pl.when(i >= 4)
        def _():
            drain_rows(ssem.at[p])

        @pl.when(i < NCHUNK)
        def _():
            pltpu.make_async_copy(
                i_hbm.at[wid, i], idx_v.at[i % 4], isem.at[p]).start()

        @pl.when(jnp.logical_and(i >= 1, i <= NCHUNK))
        def _():
            drain_idx(isem.at[1 - p])
            pltpu.make_async_copy(
                xt_hbm.at[idx_v.at[(i - 1) % 4, 0]], slot(i - 1),
                gsem.at[1 - p]).start()

        @pl.when(jnp.logical_and(i >= 2, i <= NCHUNK + 1))
        def _():
            drain_rows(gsem.at[p])
            pltpu.make_async_copy(
                slot(i - 2), acc_sh.at[idx_v.at[(i - 2) % 4, 1]],
                ssem.at[p]).start(add=True)

        return _

    lax.fori_loop(0, NCHUNK + 2, step, None)
    drain_rows(ssem.at[NCHUNK % 2])
    drain_rows(ssem.at[(NCHUNK + 1) % 2])

    plsc.subcore_barrier()

    pltpu.sync_copy(acc_sh.at[pl.ds(row0, ROWS_PER_TILE)],
                    out_hbm.at[cid].at[pl.ds(row0, ROWS_PER_TILE)])


@functools.cache
def _sc_agg():
    return pl.kernel(
        _sc_agg_body,
        out_type=jax.ShapeDtypeStruct((NC, NPAD, D), jnp.float32),
        mesh=plsc.VectorSubcoreMesh(core_axis_name="c", subcore_axis_name="s",
                                    num_cores=NC, num_subcores=NS),
        scratch_types=[
            pltpu.VMEM((4, 2, K), jnp.int32),
            pltpu.VMEM((4 * K, D), jnp.float32),
            pltpu.VMEM_SHARED((NPAD, D), jnp.float32),
            pltpu.SemaphoreType.DMA((2,)),
            pltpu.SemaphoreType.DMA((2,)),
            pltpu.SemaphoreType.DMA((2,)),
        ],
    )


@jax.jit
def kernel(x, adj, weight, bias):
    blk = 1000
    grid = N // blk

    xt = pl.pallas_call(
        _tc_pre_body,
        grid=(grid,),
        in_specs=[
            pl.BlockSpec((blk, D), lambda i: (i, 0)),
            pl.BlockSpec((D, D), lambda i: (0, 0)),
            pl.BlockSpec((1, D), lambda i: (0, 0)),
        ],
        out_specs=pl.BlockSpec((blk, D), lambda i: (i, 0)),
        out_shape=jax.ShapeDtypeStruct((N, D), jnp.float32),
    )(x, weight, bias.reshape(1, D))

    s3 = adj[0].astype(jnp.int32).reshape(NW, NCHUNK, K)
    r3 = adj[1].astype(jnp.int32).reshape(NW, NCHUNK, K)
    i4 = jnp.stack([s3, r3], axis=2)
    zeros = jnp.zeros((ROWS_PER_TILE, D), jnp.float32)

    partials = _sc_agg()(xt, i4, zeros)

    out = pl.pallas_call(
        _tc_post_body,
        grid=(grid,),
        in_specs=[pl.BlockSpec((NC, blk, D), lambda i: (0, i, 0))],
        out_specs=pl.BlockSpec((blk, D), lambda i: (i, 0)),
        out_shape=jax.ShapeDtypeStruct((N, D), jnp.float32),
    )(partials)

    return (out, adj)


# speedup vs baseline: 9.6253x; 1.0107x over previous
import functools

import jax
import jax.numpy as jnp
from jax import lax
from jax.experimental import pallas as pl
from jax.experimental.pallas import tpu as pltpu
from jax.experimental.pallas import tpu_sc as plsc

MIN_NORM = 1e-15
EPS = 4e-3
N, E, D = 10000, 320000, 128
C = 1.0

NC, NS = 2, 16
NW = NC * NS
K = 80
NCHUNK = 125
EPW = NCHUNK * K
EPAD = NW * EPW
ROWS_PER_TILE = 632
NPAD = NS * ROWS_PER_TILE


def _artanh(x):
    x = jnp.clip(x, -1.0 + 1e-7, 1.0 - 1e-7)
    return 0.5 * (jnp.log1p(x) - jnp.log1p(-x))


def _norm(x):
    return jnp.clip(
        jnp.sqrt(jnp.sum(x * x, axis=-1, keepdims=True)), MIN_NORM, None)


def _proj(x):
    norm = _norm(x)
    maxnorm = (1.0 - EPS)
    return jnp.where(norm > maxnorm, x / norm * maxnorm, x)


def _expmap0(u):
    u_norm = _norm(u)
    return jnp.tanh(u_norm) * u / u_norm


def _logmap0(p):
    p_norm = _norm(p)
    return _artanh(p_norm) * p / p_norm


def _mobius_add(x, y):
    x2 = jnp.sum(x * x, axis=-1, keepdims=True)
    y2 = jnp.sum(y * y, axis=-1, keepdims=True)
    xy = jnp.sum(x * y, axis=-1, keepdims=True)
    num = (1.0 + 2.0 * xy + y2) * x + (1.0 - x2) * y
    denom = 1.0 + 2.0 * xy + x2 * y2
    return num / jnp.clip(denom, MIN_NORM, None)


def _tc_pre_body(x_ref, w_ref, b_ref, o_ref):
    x = x_ref[...]
    w = w_ref[...]
    mx = lax.dot_general(
        x, w, (((1,), (1,)), ((), ())),
        preferred_element_type=jnp.float32,
        precision=lax.Precision.HIGHEST)
    x_norm = _norm(x)
    mx_norm = _norm(mx)
    res_c = jnp.tanh(mx_norm / x_norm * _artanh(x_norm)) * mx / mx_norm
    cond = jnp.all(mx == 0.0, axis=-1, keepdims=True)
    res = _proj(jnp.where(cond, 0.0, res_c))
    hyp_bias = _proj(_expmap0(b_ref[...]))
    res = _proj(_mobius_add(res, hyp_bias))
    o_ref[...] = _logmap0(res)


def _tc_post_body(p_ref, o_ref):
    agg = p_ref[0] + p_ref[1]
    h = _proj(_expmap0(agg))
    xt2 = jax.nn.relu(_logmap0(h))
    o_ref[...] = _proj(_expmap0(xt2))


def _sc_agg_body(xt_hbm, i_hbm, z_hbm, out_hbm,
                 idx_v, rows_v, acc_sh, gsem, ssem, isem):
    cid = lax.axis_index("c")
    sid = lax.axis_index("s")
    wid = sid * NC + cid

    row0 = sid * ROWS_PER_TILE
    pltpu.sync_copy(z_hbm.at[pl.ds(0, ROWS_PER_TILE)],
                    acc_sh.at[pl.ds(row0, ROWS_PER_TILE)])

    plsc.subcore_barrier()

    def slot(i):
        return rows_v.at[pl.ds((i % 4) * K, K)]

    def drain_rows(sem):
        pltpu.make_async_copy(xt_hbm.at[pl.ds(0, K)],
                              rows_v.at[pl.ds(0, K)], sem).wait()

    def drain_idx(sem):
        pltpu.make_async_copy(i_hbm.at[wid, 0], idx_v.at[0], sem).wait()

    def step(i, _):
        p = i % 2

        @pl.when(i >= 5)
        def _():
            drain_rows(ssem.at[1 - p])

        @pl.when(i < NCHUNK)
        def _():
            pltpu.make_async_copy(
                i_hbm.at[wid, i], idx_v.at[i % 8], isem.at[p]).start()

        @pl.when(jnp.logical_and(i >= 1, i <= NCHUNK))
        def _():
            drain_idx(isem.at[1 - p])
            pltpu.make_async_copy(
                xt_hbm.at[idx_v.at[(i - 1) % 8, 0]], slot(i - 1),
                gsem.at[(i - 1) % 4]).start()

        @pl.when(jnp.logical_and(i >= 3, i <= NCHUNK + 2))
        def _():
            drain_rows(gsem.at[(i - 3) % 4])
            pltpu.make_async_copy(
                slot(i - 3), acc_sh.at[idx_v.at[(i - 3) % 8, 1]],
                ssem.at[(i - 3) % 2]).start(add=True)

        return _

    lax.fori_loop(0, NCHUNK + 3, step, None)
    drain_rows(ssem.at[NCHUNK % 2])
    drain_rows(ssem.at[(NCHUNK + 1) % 2])

    plsc.subcore_barrier()

    pltpu.sync_copy(acc_sh.at[pl.ds(row0, ROWS_PER_TILE)],
                    out_hbm.at[cid].at[pl.ds(row0, ROWS_PER_TILE)])


@functools.cache
def _sc_agg():
    return pl.kernel(
        _sc_agg_body,
        out_type=jax.ShapeDtypeStruct((NC, NPAD, D), jnp.float32),
        mesh=plsc.VectorSubcoreMesh(core_axis_name="c", subcore_axis_name="s",
                                    num_cores=NC, num_subcores=NS),
        scratch_types=[
            pltpu.VMEM((8, 2, K), jnp.int32),
            pltpu.VMEM((4 * K, D), jnp.float32),
            pltpu.VMEM_SHARED((NPAD, D), jnp.float32),
            pltpu.SemaphoreType.DMA((4,)),
            pltpu.SemaphoreType.DMA((2,)),
            pltpu.SemaphoreType.DMA((2,)),
        ],
    )


@jax.jit
def kernel(x, adj, weight, bias):
    blk = 1000
    grid = N // blk

    xt = pl.pallas_call(
        _tc_pre_body,
        grid=(grid,),
        in_specs=[
            pl.BlockSpec((blk, D), lambda i: (i, 0)),
            pl.BlockSpec((D, D), lambda i: (0, 0)),
            pl.BlockSpec((1, D), lambda i: (0, 0)),
        ],
        out_specs=pl.BlockSpec((blk, D), lambda i: (i, 0)),
        out_shape=jax.ShapeDtypeStruct((N, D), jnp.float32),
    )(x, weight, bias.reshape(1, D))

    s3 = adj[0].astype(jnp.int32).reshape(NW, NCHUNK, K)
    r3 = adj[1].astype(jnp.int32).reshape(NW, NCHUNK, K)
    i4 = jnp.stack([s3, r3], axis=2)
    zeros = jnp.zeros((ROWS_PER_TILE, D), jnp.float32)

    partials = _sc_agg()(xt, i4, zeros)

    out = pl.pallas_call(
        _tc_post_body,
        grid=(grid,),
        in_specs=[pl.BlockSpec((NC, blk, D), lambda i: (0, i, 0))],
        out_specs=pl.BlockSpec((blk, D), lambda i: (i, 0)),
        out_shape=jax.ShapeDtypeStruct((N, D), jnp.float32),
    )(partials)

    return (out, adj)


# speedup vs baseline: 9.8767x; 1.0261x over previous
import functools

import jax
import jax.numpy as jnp
from jax import lax
from jax.experimental import pallas as pl
from jax.experimental.pallas import tpu as pltpu
from jax.experimental.pallas import tpu_sc as plsc

MIN_NORM = 1e-15
EPS = 4e-3
N, E, D = 10000, 320000, 128
C = 1.0

NC, NS = 2, 16
NW = NC * NS
K = 112
NCHUNK = 90
EPW = NCHUNK * K
EPAD = NW * EPW
ROWS_PER_TILE = 632
NPAD = NS * ROWS_PER_TILE


def _artanh(x):
    x = jnp.clip(x, -1.0 + 1e-7, 1.0 - 1e-7)
    return 0.5 * (jnp.log1p(x) - jnp.log1p(-x))


def _norm(x):
    return jnp.clip(
        jnp.sqrt(jnp.sum(x * x, axis=-1, keepdims=True)), MIN_NORM, None)


def _proj(x):
    norm = _norm(x)
    maxnorm = (1.0 - EPS)
    return jnp.where(norm > maxnorm, x / norm * maxnorm, x)


def _expmap0(u):
    u_norm = _norm(u)
    return jnp.tanh(u_norm) * u / u_norm


def _logmap0(p):
    p_norm = _norm(p)
    return _artanh(p_norm) * p / p_norm


def _mobius_add(x, y):
    x2 = jnp.sum(x * x, axis=-1, keepdims=True)
    y2 = jnp.sum(y * y, axis=-1, keepdims=True)
    xy = jnp.sum(x * y, axis=-1, keepdims=True)
    num = (1.0 + 2.0 * xy + y2) * x + (1.0 - x2) * y
    denom = 1.0 + 2.0 * xy + x2 * y2
    return num / jnp.clip(denom, MIN_NORM, None)


def _tc_pre_body(x_ref, w_ref, b_ref, o_ref):
    x = x_ref[...]
    w = w_ref[...]
    mx = lax.dot_general(
        x, w, (((1,), (1,)), ((), ())),
        preferred_element_type=jnp.float32,
        precision=lax.Precision.HIGHEST)
    x_norm = _norm(x)
    mx_norm = _norm(mx)
    res_c = jnp.tanh(mx_norm / x_norm * _artanh(x_norm)) * mx / mx_norm
    cond = jnp.all(mx == 0.0, axis=-1, keepdims=True)
    res = _proj(jnp.where(cond, 0.0, res_c))
    hyp_bias = _proj(_expmap0(b_ref[...]))
    res = _proj(_mobius_add(res, hyp_bias))
    o_ref[...] = _logmap0(res)


def _tc_post_body(p_ref, o_ref):
    agg = p_ref[0] + p_ref[1]
    h = _proj(_expmap0(agg))
    xt2 = jax.nn.relu(_logmap0(h))
    o_ref[...] = _proj(_expmap0(xt2))


def _sc_agg_body(xt_hbm, i_hbm, z_hbm, out_hbm,
                 idx_v, rows_v, acc_sh, gsem, ssem, isem):
    cid = lax.axis_index("c")
    sid = lax.axis_index("s")
    wid = sid * NC + cid

    row0 = sid * ROWS_PER_TILE
    pltpu.sync_copy(z_hbm.at[pl.ds(0, ROWS_PER_TILE)],
                    acc_sh.at[pl.ds(row0, ROWS_PER_TILE)])

    plsc.subcore_barrier()

    def slot(i):
        return rows_v.at[pl.ds((i % 3) * K, K)]

    def drain_rows(sem):
        pltpu.make_async_copy(xt_hbm.at[pl.ds(0, K)],
                              rows_v.at[pl.ds(0, K)], sem).wait()

    def drain_idx(sem):
        pltpu.make_async_copy(i_hbm.at[wid, 0], idx_v.at[0], sem).wait()

    def step(i, _):
        p = i % 2

        @pl.when(i >= 4)
        def _():
            drain_rows(ssem.at[(i - 4) % 3])

        @pl.when(i < NCHUNK)
        def _():
            pltpu.make_async_copy(
                i_hbm.at[wid, i], idx_v.at[i % 4], isem.at[p]).start()

        @pl.when(jnp.logical_and(i >= 1, i <= NCHUNK))
        def _():
            drain_idx(isem.at[1 - p])
            pltpu.make_async_copy(
                xt_hbm.at[idx_v.at[(i - 1) % 4, 0]], slot(i - 1),
                gsem.at[1 - p]).start()

        @pl.when(jnp.logical_and(i >= 2, i <= NCHUNK + 1))
        def _():
            drain_rows(gsem.at[p])
            pltpu.make_async_copy(
                slot(i - 2), acc_sh.at[idx_v.at[(i - 2) % 4, 1]],
                ssem.at[(i - 2) % 3]).start(add=True)

        return _

    lax.fori_loop(0, NCHUNK + 2, step, None)
    drain_rows(ssem.at[(NCHUNK - 2) % 3])
    drain_rows(ssem.at[(NCHUNK - 1) % 3])

    plsc.subcore_barrier()

    pltpu.sync_copy(acc_sh.at[pl.ds(row0, ROWS_PER_TILE)],
                    out_hbm.at[cid].at[pl.ds(row0, ROWS_PER_TILE)])


@functools.cache
def _sc_agg():
    return pl.kernel(
        _sc_agg_body,
        out_type=jax.ShapeDtypeStruct((NC, NPAD, D), jnp.float32),
        mesh=plsc.VectorSubcoreMesh(core_axis_name="c", subcore_axis_name="s",
                                    num_cores=NC, num_subcores=NS),
        scratch_types=[
            pltpu.VMEM((4, 2, K), jnp.int32),
            pltpu.VMEM((3 * K, D), jnp.float32),
            pltpu.VMEM_SHARED((NPAD, D), jnp.float32),
            pltpu.SemaphoreType.DMA((2,)),
            pltpu.SemaphoreType.DMA((3,)),
            pltpu.SemaphoreType.DMA((2,)),
        ],
    )


@jax.jit
def kernel(x, adj, weight, bias):
    blk = 1000
    grid = N // blk

    xt = pl.pallas_call(
        _tc_pre_body,
        grid=(grid,),
        in_specs=[
            pl.BlockSpec((blk, D), lambda i: (i, 0)),
            pl.BlockSpec((D, D), lambda i: (0, 0)),
            pl.BlockSpec((1, D), lambda i: (0, 0)),
        ],
        out_specs=pl.BlockSpec((blk, D), lambda i: (i, 0)),
        out_shape=jax.ShapeDtypeStruct((N, D), jnp.float32),
    )(x, weight, bias.reshape(1, D))

    npad_e = EPAD - E
    pad_s = jnp.arange(npad_e, dtype=jnp.int32) % N
    pad_r = N + jnp.arange(npad_e, dtype=jnp.int32) % (NPAD - N)
    s3 = jnp.concatenate([adj[0].astype(jnp.int32), pad_s]).reshape(
        NW, NCHUNK, K)
    r3 = jnp.concatenate([adj[1].astype(jnp.int32), pad_r]).reshape(
        NW, NCHUNK, K)
    i4 = jnp.stack([s3, r3], axis=2)
    zeros = jnp.zeros((ROWS_PER_TILE, D), jnp.float32)

    partials = _sc_agg()(xt, i4, zeros)

    out = pl.pallas_call(
        _tc_post_body,
        grid=(grid,),
        in_specs=[pl.BlockSpec((NC, blk, D), lambda i: (0, i, 0))],
        out_specs=pl.BlockSpec((blk, D), lambda i: (i, 0)),
        out_shape=jax.ShapeDtypeStruct((N, D), jnp.float32),
    )(partials)

    return (out, adj)


# speedup vs baseline: 9.9434x; 1.0067x over previous
import functools

import jax
import jax.numpy as jnp
from jax import lax
from jax.experimental import pallas as pl
from jax.experimental.pallas import tpu as pltpu
from jax.experimental.pallas import tpu_sc as plsc

MIN_NORM = 1e-15
EPS = 4e-3
N, E, D = 10000, 320000, 128
C = 1.0

NC, NS = 2, 16
NW = NC * NS
K = 112
NCHUNK = 90
EPW = NCHUNK * K
EPAD = NW * EPW
ROWS_PER_TILE = 632
NPAD = NS * ROWS_PER_TILE


def _artanh(x):
    x = jnp.clip(x, -1.0 + 1e-7, 1.0 - 1e-7)
    return 0.5 * (jnp.log1p(x) - jnp.log1p(-x))


def _norm(x):
    return jnp.clip(
        jnp.sqrt(jnp.sum(x * x, axis=-1, keepdims=True)), MIN_NORM, None)


def _proj(x):
    norm = _norm(x)
    maxnorm = (1.0 - EPS)
    return jnp.where(norm > maxnorm, x / norm * maxnorm, x)


def _expmap0(u):
    u_norm = _norm(u)
    return jnp.tanh(u_norm) * u / u_norm


def _logmap0(p):
    p_norm = _norm(p)
    return _artanh(p_norm) * p / p_norm


def _mobius_add(x, y):
    x2 = jnp.sum(x * x, axis=-1, keepdims=True)
    y2 = jnp.sum(y * y, axis=-1, keepdims=True)
    xy = jnp.sum(x * y, axis=-1, keepdims=True)
    num = (1.0 + 2.0 * xy + y2) * x + (1.0 - x2) * y
    denom = 1.0 + 2.0 * xy + x2 * y2
    return num / jnp.clip(denom, MIN_NORM, None)


def _tc_pre_body(x_ref, w_ref, b_ref, o_ref):
    x = x_ref[...]
    w = w_ref[...]
    mx = lax.dot_general(
        x, w, (((1,), (1,)), ((), ())),
        preferred_element_type=jnp.float32,
        precision=lax.Precision.DEFAULT)
    x_norm = _norm(x)
    mx_norm = _norm(mx)
    res_c = jnp.tanh(mx_norm / x_norm * _artanh(x_norm)) * mx / mx_norm
    cond = jnp.all(mx == 0.0, axis=-1, keepdims=True)
    res = _proj(jnp.where(cond, 0.0, res_c))
    hyp_bias = _proj(_expmap0(b_ref[...]))
    res = _proj(_mobius_add(res, hyp_bias))
    o_ref[...] = _logmap0(res)


def _tc_post_body(p_ref, o_ref):
    agg = p_ref[0] + p_ref[1]
    h = _proj(_expmap0(agg))
    xt2 = jax.nn.relu(_logmap0(h))
    o_ref[...] = _proj(_expmap0(xt2))


def _sc_agg_body(xt_hbm, i_hbm, z_hbm, out_hbm,
                 idx_v, rows_v, acc_sh, gsem, ssem, isem):
    cid = lax.axis_index("c")
    sid = lax.axis_index("s")
    wid = sid * NC + cid

    row0 = sid * ROWS_PER_TILE
    pltpu.sync_copy(z_hbm.at[pl.ds(0, ROWS_PER_TILE)],
                    acc_sh.at[pl.ds(row0, ROWS_PER_TILE)])

    plsc.subcore_barrier()

    def slot(i):
        return rows_v.at[pl.ds((i % 3) * K, K)]

    def drain_rows(sem):
        pltpu.make_async_copy(xt_hbm.at[pl.ds(0, K)],
                              rows_v.at[pl.ds(0, K)], sem).wait()

    def drain_idx(sem):
        pltpu.make_async_copy(i_hbm.at[wid, 0], idx_v.at[0], sem).wait()

    def step(i, _):
        p = i % 2

        @pl.when(i >= 4)
        def _():
            drain_rows(ssem.at[(i - 4) % 3])

        @pl.when(i < NCHUNK)
        def _():
            pltpu.make_async_copy(
                i_hbm.at[wid, i], idx_v.at[i % 4], isem.at[p]).start()

        @pl.when(jnp.logical_and(i >= 1, i <= NCHUNK))
        def _():
            drain_idx(isem.at[1 - p])
            pltpu.make_async_copy(
                xt_hbm.at[idx_v.at[(i - 1) % 4, 0]], slot(i - 1),
                gsem.at[1 - p]).start()

        @pl.when(jnp.logical_and(i >= 2, i <= NCHUNK + 1))
        def _():
            drain_rows(gsem.at[p])
            pltpu.make_async_copy(
                slot(i - 2), acc_sh.at[idx_v.at[(i - 2) % 4, 1]],
                ssem.at[(i - 2) % 3]).start(add=True)

        return _

    lax.fori_loop(0, NCHUNK + 2, step, None)
    drain_rows(ssem.at[(NCHUNK - 2) % 3])
    drain_rows(ssem.at[(NCHUNK - 1) % 3])

    plsc.subcore_barrier()

    pltpu.sync_copy(acc_sh.at[pl.ds(row0, ROWS_PER_TILE)],
                    out_hbm.at[cid].at[pl.ds(row0, ROWS_PER_TILE)])


@functools.cache
def _sc_agg():
    return pl.kernel(
        _sc_agg_body,
        out_type=jax.ShapeDtypeStruct((NC, NPAD, D), jnp.float32),
        mesh=plsc.VectorSubcoreMesh(core_axis_name="c", subcore_axis_name="s",
                                    num_cores=NC, num_subcores=NS),
        scratch_types=[
            pltpu.VMEM((4, 2, K), jnp.int32),
            pltpu.VMEM((3 * K, D), jnp.float32),
            pltpu.VMEM_SHARED((NPAD, D), jnp.float32),
            pltpu.SemaphoreType.DMA((2,)),
            pltpu.SemaphoreType.DMA((3,)),
            pltpu.SemaphoreType.DMA((2,)),
        ],
    )


@jax.jit
def kernel(x, adj, weight, bias):
    blk = 2000
    grid = N // blk

    xt = pl.pallas_call(
        _tc_pre_body,
        grid=(grid,),
        in_specs=[
            pl.BlockSpec((blk, D), lambda i: (i, 0)),
            pl.BlockSpec((D, D), lambda i: (0, 0)),
            pl.BlockSpec((1, D), lambda i: (0, 0)),
        ],
        out_specs=pl.BlockSpec((blk, D), lambda i: (i, 0)),
        out_shape=jax.ShapeDtypeStruct((N, D), jnp.float32),
    )(x, weight, bias.reshape(1, D))

    npad_e = EPAD - E
    pad_s = jnp.arange(npad_e, dtype=jnp.int32) % N
    pad_r = N + jnp.arange(npad_e, dtype=jnp.int32) % (NPAD - N)
    s3 = jnp.concatenate([adj[0].astype(jnp.int32), pad_s]).reshape(
        NW, NCHUNK, K)
    r3 = jnp.concatenate([adj[1].astype(jnp.int32), pad_r]).reshape(
        NW, NCHUNK, K)
    i4 = jnp.stack([s3, r3], axis=2)
    zeros = jnp.zeros((ROWS_PER_TILE, D), jnp.float32)

    partials = _sc_agg()(xt, i4, zeros)

    out = pl.pallas_call(
        _tc_post_body,
        grid=(grid,),
        in_specs=[pl.BlockSpec((NC, blk, D), lambda i: (0, i, 0))],
        out_specs=pl.BlockSpec((blk, D), lambda i: (i, 0)),
        out_shape=jax.ShapeDtypeStruct((N, D), jnp.float32),
    )(partials)

    return (out, adj)
